# baseline (speedup 1.0000x reference)
import functools

import jax
import jax.numpy as jnp
from jax.experimental import pallas as pl
from jax.experimental.pallas import tpu as pltpu
from jax.experimental.pallas import tpu_sc as plsc


_NC = 2
_NS = 16


def _sc_wide_gather(table_w, idx):
    NI = idx.shape[0]
    WD = table_w.shape[1]
    NW = _NC * _NS
    n_per_w = NI // NW
    CU = 800
    assert n_per_w % CU == 0
    mesh = plsc.VectorSubcoreMesh(core_axis_name="c", subcore_axis_name="s")

    @functools.partial(
        pl.kernel,
        mesh=mesh,
        out_type=jax.ShapeDtypeStruct((NI, WD), table_w.dtype),
        scratch_types=[
            pltpu.VMEM((CU,), jnp.int32),
            pltpu.VMEM((CU, WD), table_w.dtype),
            pltpu.SemaphoreType.DMA,
        ],
    )
    def k(t_hbm, i_hbm, o_hbm, idx_v, rows_v, sem):
        wid = jax.lax.axis_index("s") * _NC + jax.lax.axis_index("c")
        base = wid * n_per_w

        @pl.loop(0, n_per_w, step=CU)
        def _(off):
            pltpu.sync_copy(i_hbm.at[pl.ds(base + off, CU)], idx_v)
            pltpu.async_copy(t_hbm.at[idx_v], rows_v, sem).wait()
            pltpu.sync_copy(rows_v, o_hbm.at[pl.ds(base + off, CU)])

    return k(table_w, idx)


def _sc_row_gather(table, idx):
    NI = idx.shape[0]
    D = table.shape[1]
    per_core = NI // _NC
    CHUNK = 512
    K = 64
    assert per_core % CHUNK == 0 and CHUNK % K == 0
    mesh = plsc.ScalarSubcoreMesh(axis_name="core", num_cores=_NC)

    @functools.partial(
        pl.kernel,
        mesh=mesh,
        out_type=jax.ShapeDtypeStruct((NI, D), table.dtype),
        scratch_types=[
            pltpu.SMEM((CHUNK,), jnp.int32),
            pltpu.SemaphoreType.DMA,
            pltpu.SemaphoreType.DMA,
        ],
    )
    def k(t_hbm, n_hbm, o_hbm, idx_s, sem_i, sem):
        cid = jax.lax.axis_index("core")
        base = cid * per_core

        @pl.loop(0, per_core, step=CHUNK)
        def _(coff):
            pltpu.async_copy(n_hbm.at[pl.ds(base + coff, CHUNK)], idx_s,
                             sem_i).wait()

            @pl.loop(0, CHUNK, step=K)
            def _(off):
                copies = []
                for j in range(K):
                    row = idx_s[off + j]
                    c = pltpu.make_async_copy(
                        t_hbm.at[pl.ds(row, 1)],
                        o_hbm.at[pl.ds(base + coff + off + j, 1)],
                        sem,
                    )
                    c.start()
                    copies.append(c)
                for c in copies:
                    c.wait()

    return k(table, idx)


def _prep_body(hu_ref, hr_ref, idxq_ref, code3_ref):
    BBp, L = hu_ref.shape
    WD = code3_ref.shape[2]
    hu = hu_ref[...]
    hr = hr_ref[...]
    idxq_ref[...] = jnp.transpose(hu, (1, 0)) // 4
    for l in range(L):
        c = jax.lax.bitwise_and(hu[:, l:l + 1], 3) + 4 * hr[:, l:l + 1]
        code3_ref[l, :, :] = jnp.broadcast_to(c, (BBp, WD)).astype(jnp.int8)


def _tc_prep(history_uv, history_r, WD):
    B, L = history_uv.shape
    BBp = 512
    return pl.pallas_call(
        _prep_body,
        grid=(B // BBp,),
        in_specs=[
            pl.BlockSpec((BBp, L), lambda j: (j, 0)),
            pl.BlockSpec((BBp, L), lambda j: (j, 0)),
        ],
        out_specs=[
            pl.BlockSpec((L, BBp), lambda j: (0, j)),
            pl.BlockSpec((L, BBp, WD), lambda j: (0, j, 0)),
        ],
        out_shape=[
            jax.ShapeDtypeStruct((L, B), jnp.int32),
            jax.ShapeDtypeStruct((L, B, WD), jnp.int8),
        ],
        compiler_params=pltpu.CompilerParams(dimension_semantics=("parallel",)),
    )(history_uv, history_r)


def _dot(a, b):
    return jax.lax.dot_general(
        a, b, (((1,), (0,)), ((), ())), preferred_element_type=jnp.float32
    )


def _tc_body(gu_ref, code_ref, uv_ref, r2e_ref, geW4_ref, geB_ref, w1a_ref,
             w1b_ref, w1B_ref, w2W_ref, w2B_ref, a1a_ref, a1b_ref, a1B_ref,
             a2W_ref, a2B_ref, a3W_ref, out_ref):
    L, BB, WD = gu_ref.shape
    D = geB_ref.shape[1]
    T = L * BB
    NR8 = r2e_ref.shape[0]

    ci = code_ref[...].reshape(T, WD).astype(jnp.int32)
    lane = jax.lax.broadcasted_iota(jnp.int32, (T, WD), 1)
    sel = jax.lax.bitwise_and(ci, 3)
    mask = sel == jax.lax.shift_right_logical(lane, 5)

    gw = gu_ref[...].reshape(T, WD)
    gm = jnp.where(mask, gw, 0.0).astype(jnp.bfloat16)
    e = jnp.maximum(_dot(gm, geW4_ref[...]) + geB_ref[...], 0.0)

    rp = _dot(r2e_ref[...], w1b_ref[...])
    hv8 = jax.lax.shift_right_logical(ci[:, 0:NR8], 2)
    r8 = jax.lax.broadcasted_iota(jnp.int32, (T, NR8), 1)
    oh8 = (hv8 == r8).astype(jnp.float32)

    x = jnp.maximum(_dot(e, w1a_ref[...]) + _dot(oh8, rp) + w1B_ref[...], 0.0)
    oh = jnp.maximum(_dot(x, w2W_ref[...]) + w2B_ref[...], 0.0)

    p = _dot(uv_ref[...], a1b_ref[...])
    pb = jnp.broadcast_to(p[None], (L, BB, D)).reshape(T, D)
    a1 = jnp.maximum(_dot(oh, a1a_ref[...]) + pb + a1B_ref[...], 0.0)
    a2 = jnp.maximum(_dot(a1, a2W_ref[...]) + a2B_ref[...], 0.0)

    s = jnp.sum(a2 * a3W_ref[...], axis=1, keepdims=True)
    s3 = s.reshape(L, BB, 1)
    m = jnp.max(s3, axis=0, keepdims=True)
    w = jnp.exp(s3 - m)
    den = jnp.sum(w, axis=0)
    num = jnp.sum(oh.reshape(L, BB, D) * w, axis=0)
    out_ref[...] = num / den


def _tc_compute(gu3, code3, uvrep, r2e8, geW4, geB, w1a, w1b, w1B, w2W, w2B,
                a1a, a1b, a1B, a2W, a2B, a3W):
    L, B, WD = gu3.shape
    D = geB.shape[1]
    BB = 256
    NR8 = r2e8.shape[0]

    def full(shape):
        return pl.BlockSpec(shape, lambda j: tuple(0 for _ in shape))

    in_specs = [
        pl.BlockSpec((L, BB, WD), lambda j: (0, j, 0)),
        pl.BlockSpec((L, BB, WD), lambda j: (0, j, 0)),
        pl.BlockSpec((BB, D), lambda j: (j, 0)),
        full((NR8, D)),
        full((WD, D)), full((1, D)),
        full((D, D)), full((D, D)), full((1, D)),
        full((D, D)), full((1, D)),
        full((D, D)), full((D, D)), full((1, D)),
        full((D, D)), full((1, D)),
        full((1, D)),
    ]
    return pl.pallas_call(
        _tc_body,
        grid=(B // BB,),
        in_specs=in_specs,
        out_specs=pl.BlockSpec((BB, D), lambda j: (j, 0)),
        out_shape=jax.ShapeDtypeStruct((B, D), jnp.float32),
        compiler_params=pltpu.CompilerParams(dimension_semantics=("parallel",)),
    )(gu3, code3, uvrep, r2e8, geW4, geB, w1a, w1b, w1B, w2W, w2B,
      a1a, a1b, a1B, a2W, a2B, a3W)


def kernel(nodes, history_uv, history_r, u2e, v2e, r2e, ge_W, ge_b, w_r1_W,
           w_r1_b, w_r2_W, w_r2_b, att1_W, att1_b, att2_W, att2_b, att3_W,
           att3_b):
    B, L = history_uv.shape
    D = u2e.shape[1]

    WD = 128
    u_w = jnp.concatenate([u2e[k::WD // D] for k in range(WD // D)], axis=1)
    idxq, code3 = _tc_prep(history_uv, history_r, WD)
    gu = _sc_wide_gather(u_w, idxq.reshape(B * L))
    uvrep = _sc_row_gather(v2e, nodes)
    gu3 = gu.reshape(L, B, WD)

    geW4 = jnp.concatenate([ge_W] * (WD // D), axis=0).astype(jnp.bfloat16)
    r2e8 = jnp.pad(r2e, ((0, 8 - r2e.shape[0]), (0, 0)))

    return _tc_compute(
        gu3, code3, uvrep, r2e8,
        geW4, ge_b.reshape(1, D),
        w_r1_W[:D], w_r1_W[D:], w_r1_b.reshape(1, D),
        w_r2_W, w_r2_b.reshape(1, D),
        att1_W[:D], att1_W[D:], att1_b.reshape(1, D),
        att2_W, att2_b.reshape(1, D),
        att3_W.reshape(1, D),
    )


# speedup vs baseline: 4.5671x; 4.5671x over previous
import functools

import jax
import jax.numpy as jnp
from jax.experimental import pallas as pl
from jax.experimental.pallas import tpu as pltpu
from jax.experimental.pallas import tpu_sc as plsc


_NC = 2
_NS = 16


def _sc_wide_gather(table_w, idx):
    NI = idx.shape[0]
    WD = table_w.shape[1]
    NW = _NC * _NS
    n_per_w = NI // NW
    CU = 800
    assert n_per_w % CU == 0
    mesh = plsc.VectorSubcoreMesh(core_axis_name="c", subcore_axis_name="s")

    @functools.partial(
        pl.kernel,
        mesh=mesh,
        out_type=jax.ShapeDtypeStruct((NI, WD), table_w.dtype),
        scratch_types=[
            pltpu.VMEM((CU,), jnp.int32),
            pltpu.VMEM((CU, WD), table_w.dtype),
            pltpu.SemaphoreType.DMA,
        ],
    )
    def k(t_hbm, i_hbm, o_hbm, idx_v, rows_v, sem):
        wid = jax.lax.axis_index("s") * _NC + jax.lax.axis_index("c")
        base = wid * n_per_w

        @pl.loop(0, n_per_w, step=CU)
        def _(off):
            pltpu.sync_copy(i_hbm.at[pl.ds(base + off, CU)], idx_v)
            pltpu.async_copy(t_hbm.at[idx_v], rows_v, sem).wait()
            pltpu.sync_copy(rows_v, o_hbm.at[pl.ds(base + off, CU)])

    return k(table_w, idx)


def _sc_row_gather(table, idx):
    NI = idx.shape[0]
    D = table.shape[1]
    per_core = NI // _NC
    CHUNK = 512
    K = 64
    assert per_core % CHUNK == 0 and CHUNK % K == 0
    mesh = plsc.ScalarSubcoreMesh(axis_name="core", num_cores=_NC)

    @functools.partial(
        pl.kernel,
        mesh=mesh,
        out_type=jax.ShapeDtypeStruct((NI, D), table.dtype),
        scratch_types=[
            pltpu.SMEM((CHUNK,), jnp.int32),
            pltpu.SemaphoreType.DMA,
            pltpu.SemaphoreType.DMA,
        ],
    )
    def k(t_hbm, n_hbm, o_hbm, idx_s, sem_i, sem):
        cid = jax.lax.axis_index("core")
        base = cid * per_core

        @pl.loop(0, per_core, step=CHUNK)
        def _(coff):
            pltpu.async_copy(n_hbm.at[pl.ds(base + coff, CHUNK)], idx_s,
                             sem_i).wait()

            @pl.loop(0, CHUNK, step=K)
            def _(off):
                copies = []
                for j in range(K):
                    row = idx_s[off + j]
                    c = pltpu.make_async_copy(
                        t_hbm.at[pl.ds(row, 1)],
                        o_hbm.at[pl.ds(base + coff + off + j, 1)],
                        sem,
                    )
                    c.start()
                    copies.append(c)
                for c in copies:
                    c.wait()

    return k(table, idx)


def _pack_body(x_ref, o_ref):
    RB, D = x_ref.shape
    x4 = x_ref[...].reshape(RB // 32, 4, 8, D)
    y = jnp.concatenate([x4[:, j, :, :] for j in range(4)], axis=2)
    o_ref[...] = y.reshape(RB // 4, 4 * D)


def _tc_pack(u2e):
    N, D = u2e.shape
    RB = 8000
    assert N % RB == 0 and RB % 32 == 0
    return pl.pallas_call(
        _pack_body,
        grid=(N // RB,),
        in_specs=[pl.BlockSpec((RB, D), lambda j: (j, 0))],
        out_specs=pl.BlockSpec((RB // 4, 4 * D), lambda j: (j, 0)),
        out_shape=jax.ShapeDtypeStruct((N // 4, 4 * D), jnp.float32),
        compiler_params=pltpu.CompilerParams(dimension_semantics=("parallel",)),
    )(u2e)


def _prep_body(hu_ref, hr_ref, idxq_ref, code3_ref):
    BBp, L = hu_ref.shape
    WD = code3_ref.shape[2]
    hu = hu_ref[...]
    hr = hr_ref[...]
    hut = jnp.transpose(hu, (1, 0))
    idxq_ref[...] = jax.lax.shift_left(
        jax.lax.shift_right_logical(hut, 5), 3
    ) + jax.lax.bitwise_and(hut, 7)
    for l in range(L):
        sub = jax.lax.bitwise_and(
            jax.lax.shift_right_logical(hu[:, l:l + 1], 3), 3)
        c = sub + 4 * hr[:, l:l + 1]
        code3_ref[l, :, :] = jnp.broadcast_to(c, (BBp, WD)).astype(jnp.int8)


def _tc_prep(history_uv, history_r, WD):
    B, L = history_uv.shape
    BBp = 512
    return pl.pallas_call(
        _prep_body,
        grid=(B // BBp,),
        in_specs=[
            pl.BlockSpec((BBp, L), lambda j: (j, 0)),
            pl.BlockSpec((BBp, L), lambda j: (j, 0)),
        ],
        out_specs=[
            pl.BlockSpec((L, BBp), lambda j: (0, j)),
            pl.BlockSpec((L, BBp, WD), lambda j: (0, j, 0)),
        ],
        out_shape=[
            jax.ShapeDtypeStruct((L, B), jnp.int32),
            jax.ShapeDtypeStruct((L, B, WD), jnp.int8),
        ],
        compiler_params=pltpu.CompilerParams(dimension_semantics=("parallel",)),
    )(history_uv, history_r)


def _dot(a, b):
    return jax.lax.dot_general(
        a, b, (((1,), (0,)), ((), ())), preferred_element_type=jnp.float32
    )


def _tc_body(gu_ref, code_ref, uv_ref, r2e_ref, geW4_ref, geB_ref, w1a_ref,
             w1b_ref, w1B_ref, w2W_ref, w2B_ref, a1a_ref, a1b_ref, a1B_ref,
             a2W_ref, a2B_ref, a3W_ref, out_ref):
    L, BB, WD = gu_ref.shape
    D = geB_ref.shape[1]
    T = L * BB
    NR8 = r2e_ref.shape[0]

    ci = code_ref[...].reshape(T, WD).astype(jnp.int32)
    lane = jax.lax.broadcasted_iota(jnp.int32, (T, WD), 1)
    sel = jax.lax.bitwise_and(ci, 3)
    mask = sel == jax.lax.shift_right_logical(lane, 5)

    gw = gu_ref[...].reshape(T, WD)
    gm = jnp.where(mask, gw, 0.0).astype(jnp.bfloat16)
    e = jnp.maximum(_dot(gm, geW4_ref[...]) + geB_ref[...], 0.0)

    rp = _dot(r2e_ref[...], w1b_ref[...])
    hv8 = jax.lax.shift_right_logical(ci[:, 0:NR8], 2)
    r8 = jax.lax.broadcasted_iota(jnp.int32, (T, NR8), 1)
    oh8 = (hv8 == r8).astype(jnp.float32)

    x = jnp.maximum(_dot(e, w1a_ref[...]) + _dot(oh8, rp) + w1B_ref[...], 0.0)
    oh = jnp.maximum(_dot(x, w2W_ref[...]) + w2B_ref[...], 0.0)

    p = _dot(uv_ref[...], a1b_ref[...])
    pb = jnp.broadcast_to(p[None], (L, BB, D)).reshape(T, D)
    a1 = jnp.maximum(_dot(oh, a1a_ref[...]) + pb + a1B_ref[...], 0.0)
    a2 = jnp.maximum(_dot(a1, a2W_ref[...]) + a2B_ref[...], 0.0)

    s = jnp.sum(a2 * a3W_ref[...], axis=1, keepdims=True)
    s3 = s.reshape(L, BB, 1)
    m = jnp.max(s3, axis=0, keepdims=True)
    w = jnp.exp(s3 - m)
    den = jnp.sum(w, axis=0)
    num = jnp.sum(oh.reshape(L, BB, D) * w, axis=0)
    out_ref[...] = num / den


def _tc_compute(gu3, code3, uvrep, r2e8, geW4, geB, w1a, w1b, w1B, w2W, w2B,
                a1a, a1b, a1B, a2W, a2B, a3W):
    L, B, WD = gu3.shape
    D = geB.shape[1]
    BB = 256
    NR8 = r2e8.shape[0]

    def full(shape):
        return pl.BlockSpec(shape, lambda j: tuple(0 for _ in shape))

    in_specs = [
        pl.BlockSpec((L, BB, WD), lambda j: (0, j, 0)),
        pl.BlockSpec((L, BB, WD), lambda j: (0, j, 0)),
        pl.BlockSpec((BB, D), lambda j: (j, 0)),
        full((NR8, D)),
        full((WD, D)), full((1, D)),
        full((D, D)), full((D, D)), full((1, D)),
        full((D, D)), full((1, D)),
        full((D, D)), full((D, D)), full((1, D)),
        full((D, D)), full((1, D)),
        full((1, D)),
    ]
    return pl.pallas_call(
        _tc_body,
        grid=(B // BB,),
        in_specs=in_specs,
        out_specs=pl.BlockSpec((BB, D), lambda j: (j, 0)),
        out_shape=jax.ShapeDtypeStruct((B, D), jnp.float32),
        compiler_params=pltpu.CompilerParams(dimension_semantics=("parallel",)),
    )(gu3, code3, uvrep, r2e8, geW4, geB, w1a, w1b, w1B, w2W, w2B,
      a1a, a1b, a1B, a2W, a2B, a3W)


def kernel(nodes, history_uv, history_r, u2e, v2e, r2e, ge_W, ge_b, w_r1_W,
           w_r1_b, w_r2_W, w_r2_b, att1_W, att1_b, att2_W, att2_b, att3_W,
           att3_b):
    B, L = history_uv.shape
    D = u2e.shape[1]

    WD = 128
    u_w = _tc_pack(u2e)
    idxq, code3 = _tc_prep(history_uv, history_r, WD)
    gu = _sc_wide_gather(u_w, idxq.reshape(B * L))
    uvrep = _sc_row_gather(v2e, nodes)
    gu3 = gu.reshape(L, B, WD)

    geW4 = jnp.concatenate([ge_W] * (WD // D), axis=0).astype(jnp.bfloat16)
    r2e8 = jnp.pad(r2e, ((0, 8 - r2e.shape[0]), (0, 0)))

    return _tc_compute(
        gu3, code3, uvrep, r2e8,
        geW4, ge_b.reshape(1, D),
        w_r1_W[:D], w_r1_W[D:], w_r1_b.reshape(1, D),
        w_r2_W, w_r2_b.reshape(1, D),
        att1_W[:D], att1_W[D:], att1_b.reshape(1, D),
        att2_W, att2_b.reshape(1, D),
        att3_W.reshape(1, D),
    )


# speedup vs baseline: 5.3592x; 1.1734x over previous
import functools

import jax
import jax.numpy as jnp
from jax.experimental import pallas as pl
from jax.experimental.pallas import tpu as pltpu
from jax.experimental.pallas import tpu_sc as plsc


_NC = 2
_NS = 16


def _sc_wide_gather(table_w, idx):
    NI = idx.shape[0]
    WD = table_w.shape[1]
    NW = _NC * _NS
    n_per_w = NI // NW
    CU = 800
    assert n_per_w % CU == 0
    mesh = plsc.VectorSubcoreMesh(core_axis_name="c", subcore_axis_name="s")

    @functools.partial(
        pl.kernel,
        mesh=mesh,
        out_type=jax.ShapeDtypeStruct((NI, WD), table_w.dtype),
        scratch_types=[
            pltpu.VMEM((CU,), jnp.int32),
            pltpu.VMEM((CU, WD), table_w.dtype),
            pltpu.SemaphoreType.DMA,
        ],
    )
    def k(t_hbm, i_hbm, o_hbm, idx_v, rows_v, sem):
        wid = jax.lax.axis_index("s") * _NC + jax.lax.axis_index("c")
        base = wid * n_per_w

        @pl.loop(0, n_per_w, step=CU)
        def _(off):
            pltpu.sync_copy(i_hbm.at[pl.ds(base + off, CU)], idx_v)
            pltpu.async_copy(t_hbm.at[idx_v], rows_v, sem).wait()
            pltpu.sync_copy(rows_v, o_hbm.at[pl.ds(base + off, CU)])

    return k(table_w, idx)


def _sc_row_gather(table, idx):
    NI = idx.shape[0]
    D = table.shape[1]
    per_core = NI // _NC
    CHUNK = 1024
    K = 128
    assert per_core % CHUNK == 0 and CHUNK % K == 0
    mesh = plsc.ScalarSubcoreMesh(axis_name="core", num_cores=_NC)

    @functools.partial(
        pl.kernel,
        mesh=mesh,
        out_type=jax.ShapeDtypeStruct((NI, D), table.dtype),
        scratch_types=[
            pltpu.SMEM((CHUNK,), jnp.int32),
            pltpu.SemaphoreType.DMA,
            pltpu.SemaphoreType.DMA,
        ],
    )
    def k(t_hbm, n_hbm, o_hbm, idx_s, sem_i, sem):
        cid = jax.lax.axis_index("core")
        base = cid * per_core

        @pl.loop(0, per_core, step=CHUNK)
        def _(coff):
            pltpu.async_copy(n_hbm.at[pl.ds(base + coff, CHUNK)], idx_s,
                             sem_i).wait()

            @pl.loop(0, CHUNK, step=K)
            def _(off):
                copies = []
                for j in range(K):
                    row = idx_s[off + j]
                    c = pltpu.make_async_copy(
                        t_hbm.at[pl.ds(row, 1)],
                        o_hbm.at[pl.ds(base + coff + off + j, 1)],
                        sem,
                    )
                    c.start()
                    copies.append(c)
                for c in copies:
                    c.wait()

    return k(table, idx)


def _prep_body(hu_ref, hr_ref, idxq_ref, code3_ref):
    BBp, L = hu_ref.shape
    WD = code3_ref.shape[2]
    hu = hu_ref[...]
    hr = hr_ref[...]
    hut = jnp.transpose(hu, (1, 0))
    idxq_ref[...] = jax.lax.shift_right_logical(hut, 2)
    for l in range(L):
        sub = jax.lax.bitwise_and(hu[:, l:l + 1], 3)
        c = sub + 4 * hr[:, l:l + 1]
        code3_ref[l, :, :] = jnp.broadcast_to(c, (BBp, WD)).astype(jnp.int8)


def _tc_prep(history_uv, history_r, WD):
    B, L = history_uv.shape
    BBp = 512
    return pl.pallas_call(
        _prep_body,
        grid=(B // BBp,),
        in_specs=[
            pl.BlockSpec((BBp, L), lambda j: (j, 0)),
            pl.BlockSpec((BBp, L), lambda j: (j, 0)),
        ],
        out_specs=[
            pl.BlockSpec((L, BBp), lambda j: (0, j)),
            pl.BlockSpec((L, BBp, WD), lambda j: (0, j, 0)),
        ],
        out_shape=[
            jax.ShapeDtypeStruct((L, B), jnp.int32),
            jax.ShapeDtypeStruct((L, B, WD), jnp.int8),
        ],
        compiler_params=pltpu.CompilerParams(dimension_semantics=("parallel",)),
    )(history_uv, history_r)


def _dot(a, b):
    return jax.lax.dot_general(
        a, b, (((1,), (0,)), ((), ())), preferred_element_type=jnp.float32
    )


def _tc_body(gu_ref, code_ref, uv_ref, r2e_ref, geW4_ref, geB_ref, w1a_ref,
             w1b_ref, w1B_ref, w2W_ref, w2B_ref, a1a_ref, a1b_ref, a1B_ref,
             a2W_ref, a2B_ref, a3W_ref, out_ref):
    L, BB, WD = gu_ref.shape
    D = geB_ref.shape[1]
    T = L * BB
    NR8 = r2e_ref.shape[0]

    ci = code_ref[...].reshape(T, WD).astype(jnp.int32)
    lane = jax.lax.broadcasted_iota(jnp.int32, (T, WD), 1)
    sel = jax.lax.bitwise_and(ci, 3)
    mask = sel == jax.lax.shift_right_logical(lane, 5)

    gw = gu_ref[...].reshape(T, WD)
    gm = jnp.where(mask, gw, 0.0).astype(jnp.bfloat16)
    e = jnp.maximum(_dot(gm, geW4_ref[...]) + geB_ref[...], 0.0)

    rp = _dot(r2e_ref[...], w1b_ref[...])
    hv8 = jax.lax.shift_right_logical(ci[:, 0:NR8], 2)
    r8 = jax.lax.broadcasted_iota(jnp.int32, (T, NR8), 1)
    oh8 = (hv8 == r8).astype(jnp.float32)

    x = jnp.maximum(_dot(e, w1a_ref[...]) + _dot(oh8, rp) + w1B_ref[...], 0.0)
    oh = jnp.maximum(_dot(x, w2W_ref[...]) + w2B_ref[...], 0.0)

    p = _dot(uv_ref[...], a1b_ref[...])
    pb = jnp.broadcast_to(p[None], (L, BB, D)).reshape(T, D)
    a1 = jnp.maximum(_dot(oh, a1a_ref[...]) + pb + a1B_ref[...], 0.0)
    a2 = jnp.maximum(_dot(a1, a2W_ref[...]) + a2B_ref[...], 0.0)

    s = jnp.sum(a2 * a3W_ref[...], axis=1, keepdims=True)
    s3 = s.reshape(L, BB, 1)
    m = jnp.max(s3, axis=0, keepdims=True)
    w = jnp.exp(s3 - m)
    den = jnp.sum(w, axis=0)
    num = jnp.sum(oh.reshape(L, BB, D) * w, axis=0)
    out_ref[...] = num / den


def _tc_compute(gu3, code3, uvrep, r2e8, geW4, geB, w1a, w1b, w1B, w2W, w2B,
                a1a, a1b, a1B, a2W, a2B, a3W):
    L, B, WD = gu3.shape
    D = geB.shape[1]
    BB = 256
    NR8 = r2e8.shape[0]

    def full(shape):
        return pl.BlockSpec(shape, lambda j: tuple(0 for _ in shape))

    in_specs = [
        pl.BlockSpec((L, BB, WD), lambda j: (0, j, 0)),
        pl.BlockSpec((L, BB, WD), lambda j: (0, j, 0)),
        pl.BlockSpec((BB, D), lambda j: (j, 0)),
        full((NR8, D)),
        full((WD, D)), full((1, D)),
        full((D, D)), full((D, D)), full((1, D)),
        full((D, D)), full((1, D)),
        full((D, D)), full((D, D)), full((1, D)),
        full((D, D)), full((1, D)),
        full((1, D)),
    ]
    return pl.pallas_call(
        _tc_body,
        grid=(B // BB,),
        in_specs=in_specs,
        out_specs=pl.BlockSpec((BB, D), lambda j: (j, 0)),
        out_shape=jax.ShapeDtypeStruct((B, D), jnp.float32),
        compiler_params=pltpu.CompilerParams(dimension_semantics=("parallel",)),
    )(gu3, code3, uvrep, r2e8, geW4, geB, w1a, w1b, w1B, w2W, w2B,
      a1a, a1b, a1B, a2W, a2B, a3W)


def kernel(nodes, history_uv, history_r, u2e, v2e, r2e, ge_W, ge_b, w_r1_W,
           w_r1_b, w_r2_W, w_r2_b, att1_W, att1_b, att2_W, att2_b, att3_W,
           att3_b):
    B, L = history_uv.shape
    D = u2e.shape[1]

    WD = 128
    u_w = u2e.reshape(u2e.shape[0] * D // WD, WD)
    idxq, code3 = _tc_prep(history_uv, history_r, WD)
    gu = _sc_wide_gather(u_w, idxq.reshape(B * L))
    uvrep = _sc_row_gather(v2e, nodes)
    gu3 = gu.reshape(L, B, WD)

    geW4 = jnp.concatenate([ge_W] * (WD // D), axis=0).astype(jnp.bfloat16)
    r2e8 = jnp.pad(r2e, ((0, 8 - r2e.shape[0]), (0, 0)))

    return _tc_compute(
        gu3, code3, uvrep, r2e8,
        geW4, ge_b.reshape(1, D),
        w_r1_W[:D], w_r1_W[D:], w_r1_b.reshape(1, D),
        w_r2_W, w_r2_b.reshape(1, D),
        att1_W[:D], att1_W[D:], att1_b.reshape(1, D),
        att2_W, att2_b.reshape(1, D),
        att3_W.reshape(1, D),
    )


# speedup vs baseline: 5.5560x; 1.0367x over previous
import functools

import jax
import jax.numpy as jnp
from jax.experimental import pallas as pl
from jax.experimental.pallas import tpu as pltpu
from jax.experimental.pallas import tpu_sc as plsc


_NC = 2
_NS = 16


def _sc_wide_gather(table_w, idx):
    NI = idx.shape[0]
    WD = table_w.shape[1]
    NW = _NC * _NS
    n_per_w = NI // NW
    CU = 800
    assert n_per_w % CU == 0
    mesh = plsc.VectorSubcoreMesh(core_axis_name="c", subcore_axis_name="s")

    @functools.partial(
        pl.kernel,
        mesh=mesh,
        out_type=jax.ShapeDtypeStruct((NI, WD), table_w.dtype),
        scratch_types=[
            pltpu.VMEM((CU,), jnp.int32),
            pltpu.VMEM((CU, WD), table_w.dtype),
            pltpu.SemaphoreType.DMA,
        ],
    )
    def k(t_hbm, i_hbm, o_hbm, idx_v, rows_v, sem):
        wid = jax.lax.axis_index("s") * _NC + jax.lax.axis_index("c")
        base = wid * n_per_w

        @pl.loop(0, n_per_w, step=CU)
        def _(off):
            pltpu.sync_copy(i_hbm.at[pl.ds(base + off, CU)], idx_v)
            pltpu.async_copy(t_hbm.at[idx_v], rows_v, sem).wait()
            pltpu.sync_copy(rows_v, o_hbm.at[pl.ds(base + off, CU)])

    return k(table_w, idx)


def _sc_row_gather(table, idx):
    NI = idx.shape[0]
    D = table.shape[1]
    per_core = NI // _NC
    CHUNK = 1024
    K = 128
    assert per_core % CHUNK == 0 and CHUNK % K == 0
    mesh = plsc.ScalarSubcoreMesh(axis_name="core", num_cores=_NC)

    @functools.partial(
        pl.kernel,
        mesh=mesh,
        out_type=jax.ShapeDtypeStruct((NI, D), table.dtype),
        scratch_types=[
            pltpu.SMEM((CHUNK,), jnp.int32),
            pltpu.SemaphoreType.DMA,
            pltpu.SemaphoreType.DMA,
        ],
    )
    def k(t_hbm, n_hbm, o_hbm, idx_s, sem_i, sem):
        cid = jax.lax.axis_index("core")
        base = cid * per_core

        @pl.loop(0, per_core, step=CHUNK)
        def _(coff):
            pltpu.async_copy(n_hbm.at[pl.ds(base + coff, CHUNK)], idx_s,
                             sem_i).wait()

            @pl.loop(0, CHUNK, step=K)
            def _(off):
                copies = []
                for j in range(K):
                    row = idx_s[off + j]
                    c = pltpu.make_async_copy(
                        t_hbm.at[pl.ds(row, 1)],
                        o_hbm.at[pl.ds(base + coff + off + j, 1)],
                        sem,
                    )
                    c.start()
                    copies.append(c)
                for c in copies:
                    c.wait()

    return k(table, idx)


def _prep_body(hu_ref, hr_ref, idxq_ref, code3_ref):
    BBp, L = hu_ref.shape
    WD = code3_ref.shape[2]
    hu = hu_ref[...]
    hr = hr_ref[...]
    hut = jnp.transpose(hu, (1, 0))
    idxq_ref[...] = jax.lax.shift_right_logical(hut, 2)
    for l in range(L):
        sub = jax.lax.bitwise_and(hu[:, l:l + 1], 3)
        c = sub + 4 * hr[:, l:l + 1]
        code3_ref[l, :, :] = jnp.broadcast_to(c, (BBp, WD)).astype(jnp.int8)


def _tc_prep(history_uv, history_r, WD):
    B, L = history_uv.shape
    BBp = 512
    return pl.pallas_call(
        _prep_body,
        grid=(B // BBp,),
        in_specs=[
            pl.BlockSpec((BBp, L), lambda j: (j, 0)),
            pl.BlockSpec((BBp, L), lambda j: (j, 0)),
        ],
        out_specs=[
            pl.BlockSpec((L, BBp), lambda j: (0, j)),
            pl.BlockSpec((L, BBp, WD), lambda j: (0, j, 0)),
        ],
        out_shape=[
            jax.ShapeDtypeStruct((L, B), jnp.int32),
            jax.ShapeDtypeStruct((L, B, WD), jnp.int8),
        ],
        compiler_params=pltpu.CompilerParams(dimension_semantics=("parallel",)),
    )(history_uv, history_r)


def _dot(a, b):
    return jax.lax.dot_general(
        a, b, (((1,), (0,)), ((), ())), preferred_element_type=jnp.float32
    )


def _tc_body(gu_ref, code_ref, uv_ref, r2e_ref, geW4_ref, geB_ref, w1a_ref,
             w1b_ref, w1B_ref, w2W_ref, w2B_ref, a1a_ref, a1b_ref, a1B_ref,
             a2W_ref, a2B_ref, a3W_ref, out_ref):
    L, BB, WD = gu_ref.shape
    D = geB_ref.shape[1]
    T = L * BB
    NR8 = r2e_ref.shape[0]

    ci = code_ref[...].reshape(T, WD).astype(jnp.int32)
    lane = jax.lax.broadcasted_iota(jnp.int32, (T, WD), 1)
    sel = jax.lax.bitwise_and(ci, 3)
    mask = sel == jax.lax.shift_right_logical(lane, 5)

    gw = gu_ref[...].reshape(T, WD)
    gm = jnp.where(mask, gw, 0.0).astype(jnp.bfloat16)
    e = jnp.maximum(_dot(gm, geW4_ref[...]) + geB_ref[...], 0.0)

    rp = _dot(r2e_ref[...], w1b_ref[...])
    hv8 = jax.lax.shift_right_logical(ci[:, 0:NR8], 2)
    r8 = jax.lax.broadcasted_iota(jnp.int32, (T, NR8), 1)
    oh8 = (hv8 == r8).astype(jnp.float32)

    x = jnp.maximum(_dot(e, w1a_ref[...]) + _dot(oh8, rp) + w1B_ref[...], 0.0)
    oh = jnp.maximum(_dot(x, w2W_ref[...]) + w2B_ref[...], 0.0)

    p = _dot(uv_ref[...], a1b_ref[...])
    pb = jnp.broadcast_to(p[None], (L, BB, D)).reshape(T, D)
    a1 = jnp.maximum(_dot(oh, a1a_ref[...]) + pb + a1B_ref[...], 0.0)
    a2 = jnp.maximum(_dot(a1, a2W_ref[...]) + a2B_ref[...], 0.0)

    s = jnp.sum(a2 * a3W_ref[...], axis=1, keepdims=True)
    s3 = s.reshape(L, BB, 1)
    m = jnp.max(s3, axis=0, keepdims=True)
    w = jnp.exp(s3 - m)
    den = jnp.sum(w, axis=0)
    num = jnp.sum(oh.reshape(L, BB, D) * w, axis=0)
    out_ref[...] = num / den


def _tc_compute(gu3, code3, uvrep, r2e8, geW4, geB, w1a, w1b, w1B, w2W, w2B,
                a1a, a1b, a1B, a2W, a2B, a3W, off=0):
    L, B, WD = gu3.shape
    D = geB.shape[1]
    BB = 256
    NR8 = r2e8.shape[0]
    ob = off // BB

    def full(shape):
        return pl.BlockSpec(shape, lambda j: tuple(0 for _ in shape))

    in_specs = [
        pl.BlockSpec((L, BB, WD), lambda j: (0, j, 0)),
        pl.BlockSpec((L, BB, WD), lambda j: (0, j + ob, 0)),
        pl.BlockSpec((BB, D), lambda j: (j + ob, 0)),
        full((NR8, D)),
        full((WD, D)), full((1, D)),
        full((D, D)), full((D, D)), full((1, D)),
        full((D, D)), full((1, D)),
        full((D, D)), full((D, D)), full((1, D)),
        full((D, D)), full((1, D)),
        full((1, D)),
    ]
    return pl.pallas_call(
        _tc_body,
        grid=(B // BB,),
        in_specs=in_specs,
        out_specs=pl.BlockSpec((BB, D), lambda j: (j, 0)),
        out_shape=jax.ShapeDtypeStruct((B, D), jnp.float32),
        compiler_params=pltpu.CompilerParams(dimension_semantics=("parallel",)),
    )(gu3, code3, uvrep, r2e8, geW4, geB, w1a, w1b, w1B, w2W, w2B,
      a1a, a1b, a1B, a2W, a2B, a3W)


def kernel(nodes, history_uv, history_r, u2e, v2e, r2e, ge_W, ge_b, w_r1_W,
           w_r1_b, w_r2_W, w_r2_b, att1_W, att1_b, att2_W, att2_b, att3_W,
           att3_b):
    B, L = history_uv.shape
    D = u2e.shape[1]

    WD = 128
    u_w = u2e.reshape(u2e.shape[0] * D // WD, WD)
    idxq, code3 = _tc_prep(history_uv, history_r, WD)
    uvrep = _sc_row_gather(v2e, nodes)

    geW4 = jnp.concatenate([ge_W] * (WD // D), axis=0).astype(jnp.bfloat16)
    r2e8 = jnp.pad(r2e, ((0, 8 - r2e.shape[0]), (0, 0)))

    H = B // 2
    outs = []
    for h in range(2):
        sl = slice(h * H, (h + 1) * H)
        gu_h = _sc_wide_gather(u_w, idxq[:, sl].reshape(L * H))
        outs.append(_tc_compute(
            gu_h.reshape(L, H, WD), code3, uvrep, r2e8,
            geW4, ge_b.reshape(1, D),
            w_r1_W[:D], w_r1_W[D:], w_r1_b.reshape(1, D),
            w_r2_W, w_r2_b.reshape(1, D),
            att1_W[:D], att1_W[D:], att1_b.reshape(1, D),
            att2_W, att2_b.reshape(1, D),
            att3_W.reshape(1, D),
            off=h * H,
        ))
    return jnp.concatenate(outs, axis=0)


# speedup vs baseline: 5.5567x; 1.0001x over previous
import functools

import jax
import jax.numpy as jnp
from jax.experimental import pallas as pl
from jax.experimental.pallas import tpu as pltpu
from jax.experimental.pallas import tpu_sc as plsc


_NC = 2
_NS = 16


def _sc_wide_gather(table_w, idx):
    NI = idx.shape[0]
    WD = table_w.shape[1]
    NW = _NC * _NS
    n_per_w = NI // NW
    CU = 800
    assert n_per_w % CU == 0
    mesh = plsc.VectorSubcoreMesh(core_axis_name="c", subcore_axis_name="s")

    @functools.partial(
        pl.kernel,
        mesh=mesh,
        out_type=jax.ShapeDtypeStruct((NI, WD), table_w.dtype),
        scratch_types=[
            pltpu.VMEM((CU,), jnp.int32),
            pltpu.VMEM((CU, WD), table_w.dtype),
            pltpu.SemaphoreType.DMA,
        ],
    )
    def k(t_hbm, i_hbm, o_hbm, idx_v, rows_v, sem):
        wid = jax.lax.axis_index("s") * _NC + jax.lax.axis_index("c")
        base = wid * n_per_w

        @pl.loop(0, n_per_w, step=CU)
        def _(off):
            pltpu.sync_copy(i_hbm.at[pl.ds(base + off, CU)], idx_v)
            pltpu.async_copy(t_hbm.at[idx_v], rows_v, sem).wait()
            pltpu.sync_copy(rows_v, o_hbm.at[pl.ds(base + off, CU)])

    return k(table_w, idx)


def _sc_row_gather(table, idx):
    NI = idx.shape[0]
    D = table.shape[1]
    per_core = NI // _NC
    CHUNK = 1024
    K = 128
    assert per_core % CHUNK == 0 and CHUNK % K == 0
    mesh = plsc.ScalarSubcoreMesh(axis_name="core", num_cores=_NC)

    @functools.partial(
        pl.kernel,
        mesh=mesh,
        out_type=jax.ShapeDtypeStruct((NI, D), table.dtype),
        scratch_types=[
            pltpu.SMEM((CHUNK,), jnp.int32),
            pltpu.SemaphoreType.DMA,
            pltpu.SemaphoreType.DMA,
        ],
    )
    def k(t_hbm, n_hbm, o_hbm, idx_s, sem_i, sem):
        cid = jax.lax.axis_index("core")
        base = cid * per_core

        @pl.loop(0, per_core, step=CHUNK)
        def _(coff):
            pltpu.async_copy(n_hbm.at[pl.ds(base + coff, CHUNK)], idx_s,
                             sem_i).wait()

            @pl.loop(0, CHUNK, step=K)
            def _(off):
                copies = []
                for j in range(K):
                    row = idx_s[off + j]
                    c = pltpu.make_async_copy(
                        t_hbm.at[pl.ds(row, 1)],
                        o_hbm.at[pl.ds(base + coff + off + j, 1)],
                        sem,
                    )
                    c.start()
                    copies.append(c)
                for c in copies:
                    c.wait()

    return k(table, idx)


def _prep_body(hu_ref, hr_ref, idxq_ref, code3_ref):
    BBp, L = hu_ref.shape
    WD = code3_ref.shape[2]
    hu = hu_ref[...]
    hr = hr_ref[...]
    hut = jnp.transpose(hu, (1, 0))
    idxq_ref[...] = jax.lax.shift_right_logical(hut, 2)
    for l in range(L):
        sub = jax.lax.bitwise_and(hu[:, l:l + 1], 3)
        c = sub + 4 * hr[:, l:l + 1]
        code3_ref[l, :, :] = jnp.broadcast_to(c, (BBp, WD)).astype(jnp.int8)


def _tc_prep(history_uv, history_r, WD):
    B, L = history_uv.shape
    BBp = 512
    return pl.pallas_call(
        _prep_body,
        grid=(B // BBp,),
        in_specs=[
            pl.BlockSpec((BBp, L), lambda j: (j, 0)),
            pl.BlockSpec((BBp, L), lambda j: (j, 0)),
        ],
        out_specs=[
            pl.BlockSpec((L, BBp), lambda j: (0, j)),
            pl.BlockSpec((L, BBp, WD), lambda j: (0, j, 0)),
        ],
        out_shape=[
            jax.ShapeDtypeStruct((L, B), jnp.int32),
            jax.ShapeDtypeStruct((L, B, WD), jnp.int8),
        ],
        compiler_params=pltpu.CompilerParams(dimension_semantics=("parallel",)),
    )(history_uv, history_r)


def _dot(a, b):
    return jax.lax.dot_general(
        a, b, (((1,), (0,)), ((), ())), preferred_element_type=jnp.float32
    )


def _tc_body(gu_ref, code_ref, uv_ref, r2e_ref, geW4_ref, geB_ref, w1a_ref,
             w1b_ref, w1B_ref, w2W_ref, w2B_ref, a1a_ref, a1b_ref, a1B_ref,
             a2W_ref, a2B_ref, a3W_ref, out_ref):
    L, BB, WD = gu_ref.shape
    D = geB_ref.shape[1]
    T = L * BB
    NR8 = r2e_ref.shape[0]

    ci = code_ref[...].reshape(T, WD).astype(jnp.int32)
    lane = jax.lax.broadcasted_iota(jnp.int32, (T, WD), 1)
    sel = jax.lax.bitwise_and(ci, 3)
    mask = sel == jax.lax.shift_right_logical(lane, 5)

    gw = gu_ref[...].reshape(T, WD)
    gm = jnp.where(mask, gw, 0.0).astype(jnp.bfloat16)
    e = jnp.maximum(_dot(gm, geW4_ref[...]) + geB_ref[...], 0.0)

    rp = _dot(r2e_ref[...], w1b_ref[...])
    hv8 = jax.lax.shift_right_logical(ci[:, 0:NR8], 2)
    r8 = jax.lax.broadcasted_iota(jnp.int32, (T, NR8), 1)
    oh8 = (hv8 == r8).astype(jnp.float32)

    x = jnp.maximum(_dot(e, w1a_ref[...]) + _dot(oh8, rp) + w1B_ref[...], 0.0)
    oh = jnp.maximum(_dot(x, w2W_ref[...]) + w2B_ref[...], 0.0)

    p = _dot(uv_ref[...], a1b_ref[...])
    pb = jnp.broadcast_to(p[None], (L, BB, D)).reshape(T, D)
    a1 = jnp.maximum(_dot(oh, a1a_ref[...]) + pb + a1B_ref[...], 0.0)
    a2 = jnp.maximum(_dot(a1, a2W_ref[...]) + a2B_ref[...], 0.0)

    s = jnp.sum(a2 * a3W_ref[...], axis=1, keepdims=True)
    s3 = s.reshape(L, BB, 1)
    m = jnp.max(s3, axis=0, keepdims=True)
    w = jnp.exp(s3 - m)
    den = jnp.sum(w, axis=0)
    num = jnp.sum(oh.reshape(L, BB, D) * w, axis=0)
    out_ref[...] = num / den


def _tc_compute(gu3, code3, uvrep, r2e8, geW4, geB, w1a, w1b, w1B, w2W, w2B,
                a1a, a1b, a1B, a2W, a2B, a3W, off=0):
    L, B, WD = gu3.shape
    D = geB.shape[1]
    BB = 256
    NR8 = r2e8.shape[0]
    ob = off // BB

    def full(shape):
        return pl.BlockSpec(shape, lambda j: tuple(0 for _ in shape))

    in_specs = [
        pl.BlockSpec((L, BB, WD), lambda j: (0, j, 0)),
        pl.BlockSpec((L, BB, WD), lambda j: (0, j + ob, 0)),
        pl.BlockSpec((BB, D), lambda j: (j + ob, 0)),
        full((NR8, D)),
        full((WD, D)), full((1, D)),
        full((D, D)), full((D, D)), full((1, D)),
        full((D, D)), full((1, D)),
        full((D, D)), full((D, D)), full((1, D)),
        full((D, D)), full((1, D)),
        full((1, D)),
    ]
    return pl.pallas_call(
        _tc_body,
        grid=(B // BB,),
        in_specs=in_specs,
        out_specs=pl.BlockSpec((BB, D), lambda j: (j, 0)),
        out_shape=jax.ShapeDtypeStruct((B, D), jnp.float32),
        compiler_params=pltpu.CompilerParams(dimension_semantics=("parallel",)),
    )(gu3, code3, uvrep, r2e8, geW4, geB, w1a, w1b, w1B, w2W, w2B,
      a1a, a1b, a1B, a2W, a2B, a3W)


def kernel(nodes, history_uv, history_r, u2e, v2e, r2e, ge_W, ge_b, w_r1_W,
           w_r1_b, w_r2_W, w_r2_b, att1_W, att1_b, att2_W, att2_b, att3_W,
           att3_b):
    B, L = history_uv.shape
    D = u2e.shape[1]

    WD = 128
    u_w = u2e.reshape(u2e.shape[0] * D // WD, WD)
    idxq, code3 = _tc_prep(history_uv, history_r, WD)
    uvrep = _sc_row_gather(v2e, nodes)

    geW4 = jnp.concatenate([ge_W] * (WD // D), axis=0).astype(jnp.bfloat16)
    r2e8 = jnp.pad(r2e, ((0, 8 - r2e.shape[0]), (0, 0)))

    H = B // 4
    outs = []
    for h in range(4):
        sl = slice(h * H, (h + 1) * H)
        gu_h = _sc_wide_gather(u_w, idxq[:, sl].reshape(L * H))
        outs.append(_tc_compute(
            gu_h.reshape(L, H, WD), code3, uvrep, r2e8,
            geW4, ge_b.reshape(1, D),
            w_r1_W[:D], w_r1_W[D:], w_r1_b.reshape(1, D),
            w_r2_W, w_r2_b.reshape(1, D),
            att1_W[:D], att1_W[D:], att1_b.reshape(1, D),
            att2_W, att2_b.reshape(1, D),
            att3_W.reshape(1, D),
            off=h * H,
        ))
    return jnp.concatenate(outs, axis=0)
